# baseline (speedup 1.0000x reference)
import functools

import jax
import jax.numpy as jnp
from jax import lax
from jax.experimental import pallas as pl
from jax.experimental.pallas import tpu as pltpu
from jax.experimental.pallas import tpu_sc as plsc

_N_USERS = 100000
_N_ITEMS = 100000
_EMBED = 128
_BATCH = 1024

_W = 25600
_NSEC = 4
_TAILR = _N_ITEMS - (_NSEC - 1) * _W
_MB = 128
_NBAND = _BATCH // _MB


def _make_sc_gather(V, D, B):
    info = plsc.get_sparse_core_info()
    NC, NS = info.num_cores, info.num_subcores
    NW = NC * NS
    assert B % (8 * NW) == 0
    b_per_w = B // NW
    mesh = plsc.VectorSubcoreMesh(core_axis_name="c", subcore_axis_name="s")

    @functools.partial(
        pl.kernel,
        mesh=mesh,
        out_type=jax.ShapeDtypeStruct((B, D), jnp.float32),
        scratch_types=[
            pltpu.VMEM((b_per_w,), jnp.int32),
            pltpu.VMEM((b_per_w, D), jnp.float32),
            pltpu.SemaphoreType.DMA,
        ],
    )
    def sc_gather(table_hbm, idx_hbm, out_hbm, idx_v, rows_v, sem):
        wid = lax.axis_index("s") * NC + lax.axis_index("c")
        base = wid * b_per_w
        pltpu.sync_copy(idx_hbm.at[pl.ds(base, b_per_w)], idx_v)
        pltpu.async_copy(table_hbm.at[idx_v], rows_v, sem).wait()
        pltpu.sync_copy(rows_v, out_hbm.at[pl.ds(base, b_per_w)])

    return sc_gather


_BI3 = 2000


def _mm3_body(u_ref, it_ref, o_ref):
    u = u_ref[...]
    it = it_ref[...].astype(jnp.bfloat16)
    o_ref[0] = lax.dot_general(
        u, it, (((1,), (1,)), ((), ())), preferred_element_type=jnp.float32
    ).astype(jnp.bfloat16)


def _tc_matmul3(user_rep, rep):
    nb = _N_ITEMS // _BI3
    out3 = pl.pallas_call(
        _mm3_body,
        grid=(nb,),
        in_specs=[
            pl.BlockSpec((_BATCH, _EMBED), lambda j: (0, 0)),
            pl.BlockSpec((_BI3, _EMBED), lambda j: (_N_USERS // _BI3 + j, 0)),
        ],
        out_specs=pl.BlockSpec((1, _BATCH, _BI3), lambda j: (j, 0, 0)),
        out_shape=jax.ShapeDtypeStruct((nb, _BATCH, _BI3), jnp.bfloat16),
        compiler_params=pltpu.CompilerParams(
            dimension_semantics=("arbitrary",),
        ),
    )(user_rep, rep)
    return (
        jnp.swapaxes(out3, 0, 1)
        .reshape(_BATCH, _N_ITEMS)
        .astype(jnp.float32)
    )


def _sec_copy(rep_ref, items_v, items_sem, t):
    rows = _W if t < _NSEC - 1 else _TAILR
    return pltpu.make_async_copy(
        rep_ref.at[pl.ds(_N_USERS + t * _W, rows), :],
        items_v.at[lax.rem(t, 2), pl.ds(0, rows), :],
        items_sem.at[lax.rem(t, 2)],
    )


def _mm_body(u_ref, rep_ref, o_ref, items_v, it_bf, items_sem):
    j = pl.program_id(0)
    i = pl.program_id(1)

    @pl.when(jnp.logical_and(j == 0, i == 0))
    def _():
        _sec_copy(rep_ref, items_v, items_sem, 0).start()

    @pl.when(i == 0)
    def _():
        for t in range(_NSEC):
            @pl.when(j == t)
            def _():
                _sec_copy(rep_ref, items_v, items_sem, t).wait()
                if t + 1 < _NSEC:
                    _sec_copy(rep_ref, items_v, items_sem, t + 1).start()

        slot = lax.rem(j, 2)
        it_bf[...] = items_v[slot].astype(jnp.bfloat16)

    u = u_ref[...]
    o_ref[...] = lax.dot_general(
        u, it_bf[...], (((1,), (1,)), ((), ())),
        preferred_element_type=jnp.float32,
    )


def _tc_matmul(user_rep, rep):
    return pl.pallas_call(
        _mm_body,
        grid=(_NSEC, _NBAND),
        in_specs=[
            pl.BlockSpec((_MB, _EMBED), lambda j, i: (i, 0)),
            pl.BlockSpec(memory_space=pl.ANY),
        ],
        out_specs=pl.BlockSpec((_MB, _W), lambda j, i: (i, j)),
        out_shape=jax.ShapeDtypeStruct((_BATCH, _N_ITEMS), jnp.float32),
        scratch_shapes=[
            pltpu.VMEM((2, _W, _EMBED), jnp.float32),
            pltpu.VMEM((_W, _EMBED), jnp.bfloat16),
            pltpu.SemaphoreType.DMA((2,)),
        ],
        compiler_params=pltpu.CompilerParams(
            dimension_semantics=("arbitrary", "arbitrary"),
            vmem_limit_bytes=62 * 1024 * 1024,
            flags={"xla_mosaic_use_strided_memcopy": False},
        ),
    )(user_rep, rep)


def kernel(users, rep):
    V, D = rep.shape
    gather = _make_sc_gather(V, D, _BATCH)
    user_rep = gather(rep, users.astype(jnp.int32)).astype(jnp.bfloat16)
    return _tc_matmul3(user_rep, rep)


# speedup vs baseline: 1.1847x; 1.1847x over previous
import functools

import jax
import jax.numpy as jnp
from jax import lax
from jax.experimental import pallas as pl
from jax.experimental.pallas import tpu as pltpu
from jax.experimental.pallas import tpu_sc as plsc

_N_USERS = 100000
_N_ITEMS = 100000
_EMBED = 128
_BATCH = 1024

_W = 25600
_NSEC = 4
_TAILR = _N_ITEMS - (_NSEC - 1) * _W
_MB = 128
_NBAND = _BATCH // _MB


def _make_sc_gather(V, D, B):
    info = plsc.get_sparse_core_info()
    NC, NS = info.num_cores, info.num_subcores
    NW = NC * NS
    assert B % (8 * NW) == 0
    b_per_w = B // NW
    mesh = plsc.VectorSubcoreMesh(core_axis_name="c", subcore_axis_name="s")

    @functools.partial(
        pl.kernel,
        mesh=mesh,
        out_type=jax.ShapeDtypeStruct((B, D), jnp.float32),
        scratch_types=[
            pltpu.VMEM((b_per_w,), jnp.int32),
            pltpu.VMEM((b_per_w, D), jnp.float32),
            pltpu.SemaphoreType.DMA,
        ],
    )
    def sc_gather(table_hbm, idx_hbm, out_hbm, idx_v, rows_v, sem):
        wid = lax.axis_index("s") * NC + lax.axis_index("c")
        base = wid * b_per_w
        pltpu.sync_copy(idx_hbm.at[pl.ds(base, b_per_w)], idx_v)
        pltpu.async_copy(table_hbm.at[idx_v], rows_v, sem).wait()
        pltpu.sync_copy(rows_v, out_hbm.at[pl.ds(base, b_per_w)])

    return sc_gather


def _sec_copy(rep_ref, items_v, items_sem, t):
    rows = _W if t < _NSEC - 1 else _TAILR
    return pltpu.make_async_copy(
        rep_ref.at[pl.ds(_N_USERS + t * _W, rows), :],
        items_v.at[lax.rem(t, 2), pl.ds(0, rows), :],
        items_sem.at[lax.rem(t, 2)],
    )


def _mm_body(u_ref, rep_ref, o_ref, items_v, it_bf, items_sem):
    j = pl.program_id(0)
    i = pl.program_id(1)

    @pl.when(jnp.logical_and(j == 0, i == 0))
    def _():
        _sec_copy(rep_ref, items_v, items_sem, 0).start()

    @pl.when(i == 0)
    def _():
        for t in range(_NSEC):
            @pl.when(j == t)
            def _():
                _sec_copy(rep_ref, items_v, items_sem, t).wait()
                if t + 1 < _NSEC:
                    _sec_copy(rep_ref, items_v, items_sem, t + 1).start()

        slot = lax.rem(j, 2)
        it_bf[...] = items_v[slot].astype(jnp.bfloat16)

    u = u_ref[...]
    o_ref[...] = lax.dot_general(
        u, it_bf[...], (((1,), (1,)), ((), ())),
        preferred_element_type=jnp.float32,
    )


def _tc_matmul(user_rep, rep):
    return pl.pallas_call(
        _mm_body,
        grid=(_NSEC, _NBAND),
        in_specs=[
            pl.BlockSpec((_MB, _EMBED), lambda j, i: (i, 0)),
            pl.BlockSpec(memory_space=pl.ANY),
        ],
        out_specs=pl.BlockSpec((_MB, _W), lambda j, i: (i, j)),
        out_shape=jax.ShapeDtypeStruct((_BATCH, _N_ITEMS), jnp.float32),
        scratch_shapes=[
            pltpu.VMEM((2, _W, _EMBED), jnp.float32),
            pltpu.VMEM((_W, _EMBED), jnp.bfloat16),
            pltpu.SemaphoreType.DMA((2,)),
        ],
        compiler_params=pltpu.CompilerParams(
            dimension_semantics=("arbitrary", "arbitrary"),
            vmem_limit_bytes=62 * 1024 * 1024,
        ),
    )(user_rep, rep)


def kernel(users, rep):
    V, D = rep.shape
    gather = _make_sc_gather(V, D, _BATCH)
    user_rep = gather(rep, users.astype(jnp.int32)).astype(jnp.bfloat16)
    return _tc_matmul(user_rep, rep)
